# baseline (speedup 1.0000x reference)
import jax
import jax.numpy as jnp
from jax import lax
from jax.experimental import pallas as pl
from jax.experimental.pallas import tpu as pltpu

_DROP = 0.2
_SCALE = 1.0 / (1.0 - _DROP)
_S2 = _SCALE * _SCALE

_CR = 16
_DEPTH = 8


def _body(mu4_hbm, keep4_hbm, sig_hbm,
          mu_out_hbm, sig_out_hbm,
          mu4_v, keep4_v, colf_v, rowf_v,
          in_bufs, out_bufs, small_sems, in_sems, out_sems):
    n_b, n_i = sig_hbm.shape[0], sig_hbm.shape[1]
    chunks_per_b = n_i // _CR
    n_chunks = n_b * chunks_per_b

    def sig_in_chunk(i):
        return sig_hbm.at[i // chunks_per_b,
                          pl.ds(pl.multiple_of((i % chunks_per_b) * _CR, _CR),
                                _CR)]

    def sig_out_chunk(i):
        return sig_out_hbm.at[i // chunks_per_b,
                              pl.ds(pl.multiple_of((i % chunks_per_b) * _CR,
                                                   _CR), _CR)]

    pltpu.make_async_copy(mu4_hbm, mu4_v, small_sems.at[0]).start()
    pltpu.make_async_copy(keep4_hbm, keep4_v, small_sems.at[1]).start()

    for d in range(_DEPTH):
        pltpu.make_async_copy(sig_in_chunk(d), in_bufs.at[d],
                              in_sems.at[d]).start()

    pltpu.make_async_copy(mu4_hbm, mu4_v, small_sems.at[0]).wait()
    pltpu.make_async_copy(keep4_hbm, keep4_v, small_sems.at[1]).wait()

    mu4 = mu4_v[...]
    keep4 = keep4_v[...]
    mu_scaled = mu4 * (_SCALE * keep4)
    mu4_v[...] = mu_scaled
    colf_v[...] = jnp.where(mu_scaled != 0.0, 1.0, 0.0)
    rowf_v[...] = jnp.reshape(jnp.where(mu_scaled != 0.0, _S2, 0.0),
                              rowf_v.shape)
    pltpu.make_async_copy(mu4_v, mu_out_hbm, small_sems.at[0]).start()

    def loop(i, carry):
        slot = lax.rem(i, _DEPTH)
        b = i // chunks_per_b
        ic = lax.rem(i, chunks_per_b)
        pltpu.make_async_copy(sig_in_chunk(i), in_bufs.at[slot],
                              in_sems.at[slot]).wait()

        @pl.when(i >= _DEPTH)
        def _():
            pltpu.make_async_copy(out_bufs.at[slot], sig_out_chunk(i),
                                  out_sems.at[slot]).wait()

        sig = in_bufs[slot]
        rowf = rowf_v[b, ic]
        colf = colf_v[b]
        out_bufs[slot] = sig * rowf[:, None, :] * colf[None, :, :]
        pltpu.make_async_copy(out_bufs.at[slot], sig_out_chunk(i),
                              out_sems.at[slot]).start()

        @pl.when(i + _DEPTH < n_chunks)
        def _():
            pltpu.make_async_copy(sig_in_chunk(i + _DEPTH), in_bufs.at[slot],
                                  in_sems.at[slot]).start()
        return carry

    lax.fori_loop(0, n_chunks, loop, 0)

    for d in range(_DEPTH):
        i = n_chunks - _DEPTH + d
        slot = i % _DEPTH
        pltpu.make_async_copy(out_bufs.at[slot], sig_out_chunk(i),
                              out_sems.at[slot]).wait()
    pltpu.make_async_copy(mu4_v, mu_out_hbm, small_sems.at[0]).wait()


def kernel(mu_in, Sigma_in):
    B, H, W, C = mu_in.shape
    HW = H * W
    chunks_per_b = HW // _CR
    keep = jax.random.bernoulli(jax.random.key(42), 1.0 - _DROP, mu_in.shape)
    keepf = keep.astype(jnp.float32).reshape(B, HW, C)
    mu3 = mu_in.reshape(B, HW, C)

    hbm = pl.BlockSpec(memory_space=pltpu.MemorySpace.HBM)
    mu_out3, sig_out = pl.pallas_call(
        _body,
        in_specs=[hbm] * 3,
        out_specs=[hbm, hbm],
        out_shape=[
            jax.ShapeDtypeStruct((B, HW, C), jnp.float32),
            jax.ShapeDtypeStruct((B, HW, HW, C), jnp.float32),
        ],
        scratch_shapes=[
            pltpu.VMEM((B, HW, C), jnp.float32),
            pltpu.VMEM((B, HW, C), jnp.float32),
            pltpu.VMEM((B, HW, C), jnp.float32),
            pltpu.VMEM((B, chunks_per_b, _CR, C), jnp.float32),
            pltpu.VMEM((_DEPTH, _CR, HW, C), jnp.float32),
            pltpu.VMEM((_DEPTH, _CR, HW, C), jnp.float32),
            pltpu.SemaphoreType.DMA((2,)),
            pltpu.SemaphoreType.DMA((_DEPTH,)),
            pltpu.SemaphoreType.DMA((_DEPTH,)),
        ],
    )(mu3, keepf, Sigma_in)

    return mu_out3.reshape(B, H, W, C), sig_out


# speedup vs baseline: 4.1169x; 4.1169x over previous
import jax
import jax.numpy as jnp
from jax import lax
from jax.experimental import pallas as pl
from jax.experimental.pallas import tpu as pltpu

_DROP = 0.2
_SCALE = 1.0 / (1.0 - _DROP)
_S2 = _SCALE * _SCALE

_CR = 16
_DEPTH = 8


def _body(mur_hbm, keepr_hbm, mut_hbm, keept_hbm, sig_hbm,
          mu_out_hbm, sig_out_hbm,
          mur_v, keepr_v, mut_v, keept_v, colf_v,
          in_bufs, out_bufs, small_sems, in_sems, out_sems):
    n_b = sig_hbm.shape[0]
    chunks_per_b = sig_hbm.shape[1] // _CR
    n_chunks = n_b * chunks_per_b

    def sig_in_chunk(i):
        return sig_hbm.at[i // chunks_per_b,
                          pl.ds(pl.multiple_of((i % chunks_per_b) * _CR, _CR),
                                _CR)]

    def sig_out_chunk(i):
        return sig_out_hbm.at[i // chunks_per_b,
                              pl.ds(pl.multiple_of((i % chunks_per_b) * _CR,
                                                   _CR), _CR)]

    pltpu.make_async_copy(mur_hbm, mur_v, small_sems.at[0]).start()
    pltpu.make_async_copy(keepr_hbm, keepr_v, small_sems.at[1]).start()
    pltpu.make_async_copy(mut_hbm, mut_v, small_sems.at[2]).start()
    pltpu.make_async_copy(keept_hbm, keept_v, small_sems.at[3]).start()

    for d in range(_DEPTH):
        pltpu.make_async_copy(sig_in_chunk(d), in_bufs.at[d],
                              in_sems.at[d]).start()

    pltpu.make_async_copy(mut_hbm, mut_v, small_sems.at[2]).wait()
    pltpu.make_async_copy(keept_hbm, keept_v, small_sems.at[3]).wait()

    mut = mut_v[...]
    keept = keept_v[...]
    mut_scaled = mut * (_SCALE * keept)
    mut_v[...] = mut_scaled
    colf_v[...] = jnp.where(mut_scaled != 0.0, 1.0, 0.0)
    pltpu.make_async_copy(mut_v, mu_out_hbm, small_sems.at[2]).start()

    pltpu.make_async_copy(mur_hbm, mur_v, small_sems.at[0]).wait()
    pltpu.make_async_copy(keepr_hbm, keepr_v, small_sems.at[1]).wait()

    def loop(i, carry):
        slot = lax.rem(i, _DEPTH)
        b = i // chunks_per_b
        ic = lax.rem(i, chunks_per_b)
        pltpu.make_async_copy(sig_in_chunk(i), in_bufs.at[slot],
                              in_sems.at[slot]).wait()

        @pl.when(i >= _DEPTH)
        def _():
            pltpu.make_async_copy(out_bufs.at[slot], sig_out_chunk(i),
                                  out_sems.at[slot]).wait()

        sig = in_bufs[slot]
        mur = mur_v[b, ic]
        keepr = keepr_v[b, ic]
        rowf = jnp.where(mur * keepr != 0.0, _S2, 0.0)
        colf = colf_v[b]
        out_bufs[slot] = sig * rowf[:, :, None] * colf[None, :, :]
        pltpu.make_async_copy(out_bufs.at[slot], sig_out_chunk(i),
                              out_sems.at[slot]).start()

        @pl.when(i + _DEPTH < n_chunks)
        def _():
            pltpu.make_async_copy(sig_in_chunk(i + _DEPTH), in_bufs.at[slot],
                                  in_sems.at[slot]).start()
        return carry

    lax.fori_loop(0, n_chunks, loop, 0)

    for d in range(_DEPTH):
        i = n_chunks - _DEPTH + d
        slot = i % _DEPTH
        pltpu.make_async_copy(out_bufs.at[slot], sig_out_chunk(i),
                              out_sems.at[slot]).wait()
    pltpu.make_async_copy(mut_v, mu_out_hbm, small_sems.at[2]).wait()


def kernel(mu_in, Sigma_in):
    B, H, W, C = mu_in.shape
    HW = H * W
    chunks_per_b = HW // _CR
    keep = jax.random.bernoulli(jax.random.key(42), 1.0 - _DROP, mu_in.shape)
    keepf = keep.astype(jnp.float32)
    mu_r = mu_in.reshape(B, chunks_per_b, _CR, C)
    keep_r = keepf.reshape(B, chunks_per_b, _CR, C)
    mu_t = jnp.transpose(mu_in.reshape(B, HW, C), (0, 2, 1))
    keep_t = jnp.transpose(keepf.reshape(B, HW, C), (0, 2, 1))
    sig_t = jnp.transpose(Sigma_in, (0, 1, 3, 2))

    hbm = pl.BlockSpec(memory_space=pltpu.MemorySpace.HBM)
    mu_out_t, sig_out_t = pl.pallas_call(
        _body,
        in_specs=[hbm] * 5,
        out_specs=[hbm, hbm],
        out_shape=[
            jax.ShapeDtypeStruct((B, C, HW), jnp.float32),
            jax.ShapeDtypeStruct((B, HW, C, HW), jnp.float32),
        ],
        scratch_shapes=[
            pltpu.VMEM((B, chunks_per_b, _CR, C), jnp.float32),
            pltpu.VMEM((B, chunks_per_b, _CR, C), jnp.float32),
            pltpu.VMEM((B, C, HW), jnp.float32),
            pltpu.VMEM((B, C, HW), jnp.float32),
            pltpu.VMEM((B, C, HW), jnp.float32),
            pltpu.VMEM((_DEPTH, _CR, C, HW), jnp.float32),
            pltpu.VMEM((_DEPTH, _CR, C, HW), jnp.float32),
            pltpu.SemaphoreType.DMA((4,)),
            pltpu.SemaphoreType.DMA((_DEPTH,)),
            pltpu.SemaphoreType.DMA((_DEPTH,)),
        ],
    )(mu_r, keep_r, mu_t, keep_t, sig_t)

    mu_out = jnp.transpose(mu_out_t, (0, 2, 1)).reshape(B, H, W, C)
    sig_out = jnp.transpose(sig_out_t, (0, 1, 3, 2))
    return mu_out, sig_out


# speedup vs baseline: 4.4846x; 1.0893x over previous
import jax
import jax.numpy as jnp
from jax import lax
from jax.experimental import pallas as pl
from jax.experimental.pallas import tpu as pltpu

_DROP = 0.2
_SCALE = 1.0 / (1.0 - _DROP)
_S2 = _SCALE * _SCALE

_CR = 16
_DEPTH = 12

_KEEP_CACHE = {}


def _keep_masks(shape):
    if shape not in _KEEP_CACHE:
        import numpy as np
        B, H, W, C = shape
        HW = H * W
        with jax.ensure_compile_time_eval():
            keep = np.asarray(
                jax.random.bernoulli(jax.random.key(42), 1.0 - _DROP, shape)
            ).astype(np.float32)
        k3 = keep.reshape(B, HW, C)
        _KEEP_CACHE[shape] = (k3.reshape(B, HW // _CR, _CR, C),
                              np.ascontiguousarray(k3.transpose(0, 2, 1)))
    return _KEEP_CACHE[shape]


def _body(mur_hbm, keepr_hbm, mut_hbm, keept_hbm, sig_hbm,
          mu_out_hbm, sig_out_hbm,
          mur_v, keepr_v, mut_v, keept_v, colf_v,
          in_bufs, out_bufs, small_sems, in_sems, out_sems):
    n_b = sig_hbm.shape[0]
    chunks_per_b = sig_hbm.shape[1] // _CR
    n_chunks = n_b * chunks_per_b

    def sig_in_chunk(i):
        return sig_hbm.at[i // chunks_per_b,
                          pl.ds(pl.multiple_of((i % chunks_per_b) * _CR, _CR),
                                _CR)]

    def sig_out_chunk(i):
        return sig_out_hbm.at[i // chunks_per_b,
                              pl.ds(pl.multiple_of((i % chunks_per_b) * _CR,
                                                   _CR), _CR)]

    pltpu.make_async_copy(mur_hbm, mur_v, small_sems.at[0]).start()
    pltpu.make_async_copy(keepr_hbm, keepr_v, small_sems.at[1]).start()
    pltpu.make_async_copy(mut_hbm, mut_v, small_sems.at[2]).start()
    pltpu.make_async_copy(keept_hbm, keept_v, small_sems.at[3]).start()

    for d in range(_DEPTH):
        pltpu.make_async_copy(sig_in_chunk(d), in_bufs.at[d],
                              in_sems.at[d]).start()

    pltpu.make_async_copy(mut_hbm, mut_v, small_sems.at[2]).wait()
    pltpu.make_async_copy(keept_hbm, keept_v, small_sems.at[3]).wait()

    mut = mut_v[...]
    keept = keept_v[...]
    mut_scaled = mut * (_SCALE * keept)
    mut_v[...] = mut_scaled
    colf_v[...] = jnp.where(mut_scaled != 0.0, 1.0, 0.0)
    pltpu.make_async_copy(mut_v, mu_out_hbm, small_sems.at[2]).start()

    pltpu.make_async_copy(mur_hbm, mur_v, small_sems.at[0]).wait()
    pltpu.make_async_copy(keepr_hbm, keepr_v, small_sems.at[1]).wait()

    def loop(i, carry):
        slot = lax.rem(i, _DEPTH)
        b = i // chunks_per_b
        ic = lax.rem(i, chunks_per_b)
        pltpu.make_async_copy(sig_in_chunk(i), in_bufs.at[slot],
                              in_sems.at[slot]).wait()

        @pl.when(i >= _DEPTH)
        def _():
            pltpu.make_async_copy(out_bufs.at[slot], sig_out_chunk(i),
                                  out_sems.at[slot]).wait()

        sig = in_bufs[slot]
        mur = mur_v[b, ic]
        keepr = keepr_v[b, ic]
        rowf = jnp.where(mur * keepr != 0.0, _S2, 0.0)
        colf = colf_v[b]
        out_bufs[slot] = sig * rowf[:, :, None] * colf[None, :, :]
        pltpu.make_async_copy(out_bufs.at[slot], sig_out_chunk(i),
                              out_sems.at[slot]).start()

        @pl.when(i + _DEPTH < n_chunks)
        def _():
            pltpu.make_async_copy(sig_in_chunk(i + _DEPTH), in_bufs.at[slot],
                                  in_sems.at[slot]).start()
        return carry

    lax.fori_loop(0, n_chunks, loop, 0)

    for d in range(_DEPTH):
        i = n_chunks - _DEPTH + d
        slot = i % _DEPTH
        pltpu.make_async_copy(out_bufs.at[slot], sig_out_chunk(i),
                              out_sems.at[slot]).wait()
    pltpu.make_async_copy(mut_v, mu_out_hbm, small_sems.at[2]).wait()


def kernel(mu_in, Sigma_in):
    B, H, W, C = mu_in.shape
    HW = H * W
    chunks_per_b = HW // _CR
    keep_r_np, keep_t_np = _keep_masks((B, H, W, C))
    keep_r = jnp.asarray(keep_r_np)
    keep_t = jnp.asarray(keep_t_np)
    mu_r = mu_in.reshape(B, chunks_per_b, _CR, C)
    mu_t = jnp.transpose(mu_in.reshape(B, HW, C), (0, 2, 1))
    sig_t = jnp.transpose(Sigma_in, (0, 1, 3, 2))

    hbm = pl.BlockSpec(memory_space=pltpu.MemorySpace.HBM)
    mu_out_t, sig_out_t = pl.pallas_call(
        _body,
        in_specs=[hbm] * 5,
        out_specs=[hbm, hbm],
        out_shape=[
            jax.ShapeDtypeStruct((B, C, HW), jnp.float32),
            jax.ShapeDtypeStruct((B, HW, C, HW), jnp.float32),
        ],
        scratch_shapes=[
            pltpu.VMEM((B, chunks_per_b, _CR, C), jnp.float32),
            pltpu.VMEM((B, chunks_per_b, _CR, C), jnp.float32),
            pltpu.VMEM((B, C, HW), jnp.float32),
            pltpu.VMEM((B, C, HW), jnp.float32),
            pltpu.VMEM((B, C, HW), jnp.float32),
            pltpu.VMEM((_DEPTH, _CR, C, HW), jnp.float32),
            pltpu.VMEM((_DEPTH, _CR, C, HW), jnp.float32),
            pltpu.SemaphoreType.DMA((4,)),
            pltpu.SemaphoreType.DMA((_DEPTH,)),
            pltpu.SemaphoreType.DMA((_DEPTH,)),
        ],
    )(mu_r, keep_r, mu_t, keep_t, sig_t)

    mu_out = jnp.transpose(mu_out_t, (0, 2, 1)).reshape(B, H, W, C)
    sig_out = jnp.transpose(sig_out_t, (0, 1, 3, 2))
    return mu_out, sig_out


# speedup vs baseline: 4.5074x; 1.0051x over previous
import jax
import jax.numpy as jnp
from jax import lax
from jax.experimental import pallas as pl
from jax.experimental.pallas import tpu as pltpu

_DROP = 0.2
_SCALE = 1.0 / (1.0 - _DROP)
_S2 = _SCALE * _SCALE

_CR = 16
_DEPTH = 16

_KEEP_CACHE = {}


def _keep_masks(shape):
    if shape not in _KEEP_CACHE:
        import numpy as np
        B, H, W, C = shape
        HW = H * W
        with jax.ensure_compile_time_eval():
            keep = np.asarray(
                jax.random.bernoulli(jax.random.key(42), 1.0 - _DROP, shape)
            ).astype(np.float32)
        k3 = keep.reshape(B, HW, C)
        _KEEP_CACHE[shape] = (k3.reshape(B, HW // _CR, _CR, C),
                              np.ascontiguousarray(k3.transpose(0, 2, 1)))
    return _KEEP_CACHE[shape]


def _body(mur_hbm, keepr_hbm, mut_hbm, keept_hbm, sig_hbm,
          mu_out_hbm, sig_out_hbm,
          mur_v, keepr_v, mut_v, keept_v, colf_v,
          in_bufs, out_bufs, small_sems, in_sems, out_sems):
    n_b = sig_hbm.shape[0]
    chunks_per_b = sig_hbm.shape[1] // _CR
    n_chunks = n_b * chunks_per_b

    def sig_in_chunk(i):
        return sig_hbm.at[i // chunks_per_b,
                          pl.ds(pl.multiple_of((i % chunks_per_b) * _CR, _CR),
                                _CR)]

    def sig_out_chunk(i):
        return sig_out_hbm.at[i // chunks_per_b,
                              pl.ds(pl.multiple_of((i % chunks_per_b) * _CR,
                                                   _CR), _CR)]

    pltpu.make_async_copy(mur_hbm, mur_v, small_sems.at[0]).start()
    pltpu.make_async_copy(keepr_hbm, keepr_v, small_sems.at[1]).start()
    pltpu.make_async_copy(mut_hbm, mut_v, small_sems.at[2]).start()
    pltpu.make_async_copy(keept_hbm, keept_v, small_sems.at[3]).start()

    for d in range(_DEPTH):
        pltpu.make_async_copy(sig_in_chunk(d), in_bufs.at[d],
                              in_sems.at[d]).start()

    pltpu.make_async_copy(mut_hbm, mut_v, small_sems.at[2]).wait()
    pltpu.make_async_copy(keept_hbm, keept_v, small_sems.at[3]).wait()

    mut = mut_v[...]
    keept = keept_v[...]
    mut_scaled = mut * (_SCALE * keept)
    mut_v[...] = mut_scaled
    colf_v[...] = jnp.where(mut_scaled != 0.0, 1.0, 0.0)
    pltpu.make_async_copy(mut_v, mu_out_hbm, small_sems.at[2]).start()

    pltpu.make_async_copy(mur_hbm, mur_v, small_sems.at[0]).wait()
    pltpu.make_async_copy(keepr_hbm, keepr_v, small_sems.at[1]).wait()

    def loop(i, carry):
        slot = lax.rem(i, _DEPTH)
        b = i // chunks_per_b
        ic = lax.rem(i, chunks_per_b)
        pltpu.make_async_copy(sig_in_chunk(i), in_bufs.at[slot],
                              in_sems.at[slot]).wait()

        @pl.when(i >= _DEPTH)
        def _():
            pltpu.make_async_copy(out_bufs.at[slot], sig_out_chunk(i),
                                  out_sems.at[slot]).wait()

        sig = in_bufs[slot]
        mur = mur_v[b, ic]
        keepr = keepr_v[b, ic]
        rowf = jnp.where(mur * keepr != 0.0, _S2, 0.0)
        colf = colf_v[b]
        out_bufs[slot] = sig * rowf[:, :, None] * colf[None, :, :]
        pltpu.make_async_copy(out_bufs.at[slot], sig_out_chunk(i),
                              out_sems.at[slot]).start()

        @pl.when(i + _DEPTH < n_chunks)
        def _():
            pltpu.make_async_copy(sig_in_chunk(i + _DEPTH), in_bufs.at[slot],
                                  in_sems.at[slot]).start()
        return carry

    lax.fori_loop(0, n_chunks, loop, 0)

    for d in range(_DEPTH):
        i = n_chunks - _DEPTH + d
        slot = i % _DEPTH
        pltpu.make_async_copy(out_bufs.at[slot], sig_out_chunk(i),
                              out_sems.at[slot]).wait()
    pltpu.make_async_copy(mut_v, mu_out_hbm, small_sems.at[2]).wait()


def kernel(mu_in, Sigma_in):
    B, H, W, C = mu_in.shape
    HW = H * W
    chunks_per_b = HW // _CR
    keep_r_np, keep_t_np = _keep_masks((B, H, W, C))
    keep_r = jnp.asarray(keep_r_np)
    keep_t = jnp.asarray(keep_t_np)
    mu_r = mu_in.reshape(B, chunks_per_b, _CR, C)
    mu_t = jnp.transpose(mu_in.reshape(B, HW, C), (0, 2, 1))
    sig_t = jnp.transpose(Sigma_in, (0, 1, 3, 2))

    hbm = pl.BlockSpec(memory_space=pltpu.MemorySpace.HBM)
    mu_out_t, sig_out_t = pl.pallas_call(
        _body,
        in_specs=[hbm] * 5,
        out_specs=[hbm, hbm],
        out_shape=[
            jax.ShapeDtypeStruct((B, C, HW), jnp.float32),
            jax.ShapeDtypeStruct((B, HW, C, HW), jnp.float32),
        ],
        scratch_shapes=[
            pltpu.VMEM((B, chunks_per_b, _CR, C), jnp.float32),
            pltpu.VMEM((B, chunks_per_b, _CR, C), jnp.float32),
            pltpu.VMEM((B, C, HW), jnp.float32),
            pltpu.VMEM((B, C, HW), jnp.float32),
            pltpu.VMEM((B, C, HW), jnp.float32),
            pltpu.VMEM((_DEPTH, _CR, C, HW), jnp.float32),
            pltpu.VMEM((_DEPTH, _CR, C, HW), jnp.float32),
            pltpu.SemaphoreType.DMA((4,)),
            pltpu.SemaphoreType.DMA((_DEPTH,)),
            pltpu.SemaphoreType.DMA((_DEPTH,)),
        ],
    )(mu_r, keep_r, mu_t, keep_t, sig_t)

    mu_out = jnp.transpose(mu_out_t, (0, 2, 1)).reshape(B, H, W, C)
    sig_out = jnp.transpose(sig_out_t, (0, 1, 3, 2))
    return mu_out, sig_out


# speedup vs baseline: 4.6507x; 1.0318x over previous
import jax
import jax.numpy as jnp
from jax import lax
from jax.experimental import pallas as pl
from jax.experimental.pallas import tpu as pltpu

_DROP = 0.2
_SCALE = 1.0 / (1.0 - _DROP)
_S2 = _SCALE * _SCALE

_CR = 16
_DEPTH = 16

_KEEP_CACHE = {}


def _keep_masks(shape):
    if shape not in _KEEP_CACHE:
        import numpy as np
        B, H, W, C = shape
        HW = H * W
        with jax.ensure_compile_time_eval():
            keep = np.asarray(
                jax.random.bernoulli(jax.random.key(42), 1.0 - _DROP, shape)
            ).astype(np.float32)
        k3 = keep.reshape(B, HW, C)
        _KEEP_CACHE[shape] = (k3.reshape(B, HW // _CR, _CR, C) * _SCALE,
                              np.ascontiguousarray(k3.transpose(0, 2, 1))
                              * _SCALE)
    return _KEEP_CACHE[shape]


def _body(mur_hbm, keepr_hbm, mut_hbm, keept_hbm, sig_hbm,
          mu_out_hbm, sig_out_hbm,
          mur_v, keepr_v, mut_v, keept_v, colf_v,
          in_bufs, out_bufs, small_sems, in_sems, out_sems):
    n_b = sig_hbm.shape[0]
    chunks_per_b = sig_hbm.shape[1] // _CR
    n_chunks = n_b * chunks_per_b

    def sig_in_chunk(i):
        return sig_hbm.at[i // chunks_per_b,
                          pl.ds(pl.multiple_of((i % chunks_per_b) * _CR, _CR),
                                _CR)]

    def sig_out_chunk(i):
        return sig_out_hbm.at[i // chunks_per_b,
                              pl.ds(pl.multiple_of((i % chunks_per_b) * _CR,
                                                   _CR), _CR)]

    pltpu.make_async_copy(mur_hbm, mur_v, small_sems.at[0]).start()
    pltpu.make_async_copy(keepr_hbm, keepr_v, small_sems.at[1]).start()
    pltpu.make_async_copy(mut_hbm, mut_v, small_sems.at[2]).start()
    pltpu.make_async_copy(keept_hbm, keept_v, small_sems.at[3]).start()

    for d in range(_DEPTH):
        pltpu.make_async_copy(sig_in_chunk(d), in_bufs.at[d],
                              in_sems.at[d]).start()

    pltpu.make_async_copy(mur_hbm, mur_v, small_sems.at[0]).wait()
    pltpu.make_async_copy(keepr_hbm, keepr_v, small_sems.at[1]).wait()
    mur_v[...] = mur_v[...] * keepr_v[...]
    pltpu.make_async_copy(mur_v, mu_out_hbm, small_sems.at[0]).start()

    pltpu.make_async_copy(mut_hbm, mut_v, small_sems.at[2]).wait()
    pltpu.make_async_copy(keept_hbm, keept_v, small_sems.at[3]).wait()
    colf_v[...] = jnp.where(mut_v[...] * keept_v[...] != 0.0, 1.0, 0.0)

    def loop(i, carry):
        slot = lax.rem(i, _DEPTH)
        b = i // chunks_per_b
        ic = lax.rem(i, chunks_per_b)
        pltpu.make_async_copy(sig_in_chunk(i), in_bufs.at[slot],
                              in_sems.at[slot]).wait()

        @pl.when(i >= _DEPTH)
        def _():
            pltpu.make_async_copy(out_bufs.at[slot], sig_out_chunk(i),
                                  out_sems.at[slot]).wait()

        sig = in_bufs[slot]
        rowf = jnp.where(mur_v[b, ic] != 0.0, _S2, 0.0)
        colf = colf_v[b]
        out_bufs[slot] = sig * rowf[:, :, None] * colf[None, :, :]
        pltpu.make_async_copy(out_bufs.at[slot], sig_out_chunk(i),
                              out_sems.at[slot]).start()

        @pl.when(i + _DEPTH < n_chunks)
        def _():
            pltpu.make_async_copy(sig_in_chunk(i + _DEPTH), in_bufs.at[slot],
                                  in_sems.at[slot]).start()
        return carry

    lax.fori_loop(0, n_chunks, loop, 0)

    for d in range(_DEPTH):
        i = n_chunks - _DEPTH + d
        slot = i % _DEPTH
        pltpu.make_async_copy(out_bufs.at[slot], sig_out_chunk(i),
                              out_sems.at[slot]).wait()
    pltpu.make_async_copy(mur_v, mu_out_hbm, small_sems.at[0]).wait()


def kernel(mu_in, Sigma_in):
    B, H, W, C = mu_in.shape
    HW = H * W
    chunks_per_b = HW // _CR
    keep_r_np, keep_t_np = _keep_masks((B, H, W, C))
    keep_r = jnp.asarray(keep_r_np)
    keep_t = jnp.asarray(keep_t_np)
    mu_r = mu_in.reshape(B, chunks_per_b, _CR, C)
    mu_t = jnp.transpose(mu_in.reshape(B, HW, C), (0, 2, 1))
    sig_t = jnp.transpose(Sigma_in, (0, 1, 3, 2))

    hbm = pl.BlockSpec(memory_space=pltpu.MemorySpace.HBM)
    mu_out_r, sig_out_t = pl.pallas_call(
        _body,
        in_specs=[hbm] * 5,
        out_specs=[hbm, hbm],
        out_shape=[
            jax.ShapeDtypeStruct((B, chunks_per_b, _CR, C), jnp.float32),
            jax.ShapeDtypeStruct((B, HW, C, HW), jnp.float32),
        ],
        scratch_shapes=[
            pltpu.VMEM((B, chunks_per_b, _CR, C), jnp.float32),
            pltpu.VMEM((B, chunks_per_b, _CR, C), jnp.float32),
            pltpu.VMEM((B, C, HW), jnp.float32),
            pltpu.VMEM((B, C, HW), jnp.float32),
            pltpu.VMEM((B, C, HW), jnp.float32),
            pltpu.VMEM((_DEPTH, _CR, C, HW), jnp.float32),
            pltpu.VMEM((_DEPTH, _CR, C, HW), jnp.float32),
            pltpu.SemaphoreType.DMA((4,)),
            pltpu.SemaphoreType.DMA((_DEPTH,)),
            pltpu.SemaphoreType.DMA((_DEPTH,)),
        ],
    )(mu_r, keep_r, mu_t, keep_t, sig_t)

    mu_out = mu_out_r.reshape(B, H, W, C)
    sig_out = jnp.transpose(sig_out_t, (0, 1, 3, 2))
    return mu_out, sig_out


# speedup vs baseline: 4.6766x; 1.0056x over previous
import jax
import jax.numpy as jnp
from jax import lax
from jax.experimental import pallas as pl
from jax.experimental.pallas import tpu as pltpu

_DROP = 0.2
_SCALE = 1.0 / (1.0 - _DROP)
_S2 = _SCALE * _SCALE

_CR = 32
_DEPTH = 8

_KEEP_CACHE = {}


def _keep_masks(shape):
    if shape not in _KEEP_CACHE:
        import numpy as np
        B, H, W, C = shape
        HW = H * W
        with jax.ensure_compile_time_eval():
            keep = np.asarray(
                jax.random.bernoulli(jax.random.key(42), 1.0 - _DROP, shape)
            ).astype(np.float32)
        k3 = keep.reshape(B, HW, C)
        _KEEP_CACHE[shape] = (k3.reshape(B, HW // _CR, _CR, C) * _SCALE,
                              np.ascontiguousarray(k3.transpose(0, 2, 1))
                              * _SCALE)
    return _KEEP_CACHE[shape]


def _body(mur_hbm, keepr_hbm, mut_hbm, keept_hbm, sig_hbm,
          mu_out_hbm, sig_out_hbm,
          mur_v, keepr_v, mut_v, keept_v, colf_v,
          in_bufs, out_bufs, small_sems, in_sems, out_sems):
    n_b = sig_hbm.shape[0]
    chunks_per_b = sig_hbm.shape[1] // _CR
    n_chunks = n_b * chunks_per_b

    def sig_in_chunk(i):
        return sig_hbm.at[i // chunks_per_b,
                          pl.ds(pl.multiple_of((i % chunks_per_b) * _CR, _CR),
                                _CR)]

    def sig_out_chunk(i):
        return sig_out_hbm.at[i // chunks_per_b,
                              pl.ds(pl.multiple_of((i % chunks_per_b) * _CR,
                                                   _CR), _CR)]

    pltpu.make_async_copy(mur_hbm, mur_v, small_sems.at[0]).start()
    pltpu.make_async_copy(keepr_hbm, keepr_v, small_sems.at[1]).start()
    pltpu.make_async_copy(mut_hbm, mut_v, small_sems.at[2]).start()
    pltpu.make_async_copy(keept_hbm, keept_v, small_sems.at[3]).start()

    for d in range(_DEPTH):
        pltpu.make_async_copy(sig_in_chunk(d), in_bufs.at[d],
                              in_sems.at[d]).start()

    pltpu.make_async_copy(mur_hbm, mur_v, small_sems.at[0]).wait()
    pltpu.make_async_copy(keepr_hbm, keepr_v, small_sems.at[1]).wait()
    mur_v[...] = mur_v[...] * keepr_v[...]
    pltpu.make_async_copy(mur_v, mu_out_hbm, small_sems.at[0]).start()

    pltpu.make_async_copy(mut_hbm, mut_v, small_sems.at[2]).wait()
    pltpu.make_async_copy(keept_hbm, keept_v, small_sems.at[3]).wait()
    colf_v[...] = jnp.where(mut_v[...] * keept_v[...] != 0.0, 1.0, 0.0)

    def loop(i, carry):
        slot = lax.rem(i, _DEPTH)
        b = i // chunks_per_b
        ic = lax.rem(i, chunks_per_b)
        pltpu.make_async_copy(sig_in_chunk(i), in_bufs.at[slot],
                              in_sems.at[slot]).wait()

        @pl.when(i >= _DEPTH)
        def _():
            pltpu.make_async_copy(out_bufs.at[slot], sig_out_chunk(i),
                                  out_sems.at[slot]).wait()

        sig = in_bufs[slot]
        rowf = jnp.where(mur_v[b, ic] != 0.0, _S2, 0.0)
        colf = colf_v[b]
        out_bufs[slot] = sig * rowf[:, :, None] * colf[None, :, :]
        pltpu.make_async_copy(out_bufs.at[slot], sig_out_chunk(i),
                              out_sems.at[slot]).start()

        @pl.when(i + _DEPTH < n_chunks)
        def _():
            pltpu.make_async_copy(sig_in_chunk(i + _DEPTH), in_bufs.at[slot],
                                  in_sems.at[slot]).start()
        return carry

    lax.fori_loop(0, n_chunks, loop, 0)

    for d in range(_DEPTH):
        i = n_chunks - _DEPTH + d
        slot = i % _DEPTH
        pltpu.make_async_copy(out_bufs.at[slot], sig_out_chunk(i),
                              out_sems.at[slot]).wait()
    pltpu.make_async_copy(mur_v, mu_out_hbm, small_sems.at[0]).wait()


def kernel(mu_in, Sigma_in):
    B, H, W, C = mu_in.shape
    HW = H * W
    chunks_per_b = HW // _CR
    keep_r_np, keep_t_np = _keep_masks((B, H, W, C))
    keep_r = jnp.asarray(keep_r_np)
    keep_t = jnp.asarray(keep_t_np)
    mu_r = mu_in.reshape(B, chunks_per_b, _CR, C)
    mu_t = jnp.transpose(mu_in.reshape(B, HW, C), (0, 2, 1))
    sig_t = jnp.transpose(Sigma_in, (0, 1, 3, 2))

    hbm = pl.BlockSpec(memory_space=pltpu.MemorySpace.HBM)
    mu_out_r, sig_out_t = pl.pallas_call(
        _body,
        in_specs=[hbm] * 5,
        out_specs=[hbm, hbm],
        out_shape=[
            jax.ShapeDtypeStruct((B, chunks_per_b, _CR, C), jnp.float32),
            jax.ShapeDtypeStruct((B, HW, C, HW), jnp.float32),
        ],
        scratch_shapes=[
            pltpu.VMEM((B, chunks_per_b, _CR, C), jnp.float32),
            pltpu.VMEM((B, chunks_per_b, _CR, C), jnp.float32),
            pltpu.VMEM((B, C, HW), jnp.float32),
            pltpu.VMEM((B, C, HW), jnp.float32),
            pltpu.VMEM((B, C, HW), jnp.float32),
            pltpu.VMEM((_DEPTH, _CR, C, HW), jnp.float32),
            pltpu.VMEM((_DEPTH, _CR, C, HW), jnp.float32),
            pltpu.SemaphoreType.DMA((4,)),
            pltpu.SemaphoreType.DMA((_DEPTH,)),
            pltpu.SemaphoreType.DMA((_DEPTH,)),
        ],
    )(mu_r, keep_r, mu_t, keep_t, sig_t)

    mu_out = mu_out_r.reshape(B, H, W, C)
    sig_out = jnp.transpose(sig_out_t, (0, 1, 3, 2))
    return mu_out, sig_out


# speedup vs baseline: 4.8536x; 1.0379x over previous
import jax
import jax.numpy as jnp
from jax import lax
from jax.experimental import pallas as pl
from jax.experimental.pallas import tpu as pltpu

_DROP = 0.2
_SCALE = 1.0 / (1.0 - _DROP)
_S2 = _SCALE * _SCALE

_CR = 32
_DEPTH = 8

_KEEP_CACHE = {}


def _keep_masks(shape):
    B, H, W, C = shape
    HW = H * W
    if shape not in _KEEP_CACHE:
        import numpy as np
        try:
            with jax.ensure_compile_time_eval():
                keep = np.asarray(
                    jax.random.bernoulli(jax.random.key(42), 1.0 - _DROP,
                                         shape)
                ).astype(np.float32)
            k3 = keep.reshape(B, HW, C)
            _KEEP_CACHE[shape] = (k3.reshape(B, HW // _CR, _CR, C) * _SCALE,)
        except Exception:
            keep = jax.random.bernoulli(
                jax.random.key(42), 1.0 - _DROP, shape
            ).astype(jnp.float32)
            k3 = keep.reshape(B, HW, C)
            return (k3.reshape(B, HW // _CR, _CR, C) * _SCALE,)
    return _KEEP_CACHE[shape]


def _body(mur_hbm, keepr_hbm, sig_hbm,
          mu_out_hbm, sig_out_hbm,
          mur_v, keepr_v, colf_v,
          in_bufs, out_bufs, small_sems, in_sems, out_sems):
    n_b = sig_hbm.shape[0]
    chunks_per_b = sig_hbm.shape[1] // _CR
    n_chunks = n_b * chunks_per_b

    def sig_in_chunk(i):
        return sig_hbm.at[i // chunks_per_b,
                          pl.ds(pl.multiple_of((i % chunks_per_b) * _CR, _CR),
                                _CR)]

    def sig_out_chunk(i):
        return sig_out_hbm.at[i // chunks_per_b,
                              pl.ds(pl.multiple_of((i % chunks_per_b) * _CR,
                                                   _CR), _CR)]

    pltpu.make_async_copy(mur_hbm, mur_v, small_sems.at[0]).start()
    pltpu.make_async_copy(keepr_hbm, keepr_v, small_sems.at[1]).start()

    for d in range(_DEPTH):
        pltpu.make_async_copy(sig_in_chunk(d), in_bufs.at[d],
                              in_sems.at[d]).start()

    pltpu.make_async_copy(mur_hbm, mur_v, small_sems.at[0]).wait()
    pltpu.make_async_copy(keepr_hbm, keepr_v, small_sems.at[1]).wait()
    mur_v[...] = mur_v[...] * keepr_v[...]
    pltpu.make_async_copy(mur_v, mu_out_hbm, small_sems.at[0]).start()

    for b in range(n_b):
        nzb = jnp.where(mur_v[b].reshape(chunks_per_b * _CR, -1) != 0.0,
                        1.0, 0.0)
        colf_v[b] = nzb.swapaxes(0, 1)

    def loop(i, carry):
        slot = lax.rem(i, _DEPTH)
        b = i // chunks_per_b
        ic = lax.rem(i, chunks_per_b)
        pltpu.make_async_copy(sig_in_chunk(i), in_bufs.at[slot],
                              in_sems.at[slot]).wait()

        @pl.when(i >= _DEPTH)
        def _():
            pltpu.make_async_copy(out_bufs.at[slot], sig_out_chunk(i),
                                  out_sems.at[slot]).wait()

        sig = in_bufs[slot]
        rowf = jnp.where(mur_v[b, ic] != 0.0, _S2, 0.0)
        colf = colf_v[b]
        out_bufs[slot] = sig * rowf[:, :, None] * colf[None, :, :]
        pltpu.make_async_copy(out_bufs.at[slot], sig_out_chunk(i),
                              out_sems.at[slot]).start()

        @pl.when(i + _DEPTH < n_chunks)
        def _():
            pltpu.make_async_copy(sig_in_chunk(i + _DEPTH), in_bufs.at[slot],
                                  in_sems.at[slot]).start()
        return carry

    lax.fori_loop(0, n_chunks, loop, 0)

    for d in range(_DEPTH):
        i = n_chunks - _DEPTH + d
        slot = i % _DEPTH
        pltpu.make_async_copy(out_bufs.at[slot], sig_out_chunk(i),
                              out_sems.at[slot]).wait()
    pltpu.make_async_copy(mur_v, mu_out_hbm, small_sems.at[0]).wait()


def kernel(mu_in, Sigma_in):
    B, H, W, C = mu_in.shape
    HW = H * W
    chunks_per_b = HW // _CR
    (keep_r_np,) = _keep_masks((B, H, W, C))
    keep_r = jnp.asarray(keep_r_np)
    mu_r = mu_in.reshape(B, chunks_per_b, _CR, C)
    sig_t = jnp.transpose(Sigma_in, (0, 1, 3, 2))

    hbm = pl.BlockSpec(memory_space=pltpu.MemorySpace.HBM)
    mu_out_r, sig_out_t = pl.pallas_call(
        _body,
        in_specs=[hbm] * 3,
        out_specs=[hbm, hbm],
        out_shape=[
            jax.ShapeDtypeStruct((B, chunks_per_b, _CR, C), jnp.float32),
            jax.ShapeDtypeStruct((B, HW, C, HW), jnp.float32),
        ],
        scratch_shapes=[
            pltpu.VMEM((B, chunks_per_b, _CR, C), jnp.float32),
            pltpu.VMEM((B, chunks_per_b, _CR, C), jnp.float32),
            pltpu.VMEM((B, C, HW), jnp.float32),
            pltpu.VMEM((_DEPTH, _CR, C, HW), jnp.float32),
            pltpu.VMEM((_DEPTH, _CR, C, HW), jnp.float32),
            pltpu.SemaphoreType.DMA((2,)),
            pltpu.SemaphoreType.DMA((_DEPTH,)),
            pltpu.SemaphoreType.DMA((_DEPTH,)),
        ],
    )(mu_r, keep_r, sig_t)

    mu_out = mu_out_r.reshape(B, H, W, C)
    sig_out = jnp.transpose(sig_out_t, (0, 1, 3, 2))
    return mu_out, sig_out


# speedup vs baseline: 4.8662x; 1.0026x over previous
import jax
import jax.numpy as jnp
from jax import lax
from jax.experimental import pallas as pl
from jax.experimental.pallas import tpu as pltpu

_DROP = 0.2
_SCALE = 1.0 / (1.0 - _DROP)
_S2 = _SCALE * _SCALE

_CR = 64
_DEPTH = 4

_KEEP_CACHE = {}


def _keep_masks(shape):
    B, H, W, C = shape
    HW = H * W
    if shape not in _KEEP_CACHE:
        import numpy as np
        try:
            with jax.ensure_compile_time_eval():
                keep = np.asarray(
                    jax.random.bernoulli(jax.random.key(42), 1.0 - _DROP,
                                         shape)
                ).astype(np.float32)
            k3 = keep.reshape(B, HW, C)
            _KEEP_CACHE[shape] = (k3.reshape(B, HW // _CR, _CR, C) * _SCALE,)
        except Exception:
            keep = jax.random.bernoulli(
                jax.random.key(42), 1.0 - _DROP, shape
            ).astype(jnp.float32)
            k3 = keep.reshape(B, HW, C)
            return (k3.reshape(B, HW // _CR, _CR, C) * _SCALE,)
    return _KEEP_CACHE[shape]


def _body(mur_hbm, keepr_hbm, sig_hbm,
          mu_out_hbm, sig_out_hbm,
          mur_v, keepr_v, colf_v,
          in_bufs, out_bufs, small_sems, in_sems, out_sems):
    n_b = sig_hbm.shape[0]
    chunks_per_b = sig_hbm.shape[1] // _CR
    n_chunks = n_b * chunks_per_b

    def sig_in_chunk(i):
        return sig_hbm.at[i // chunks_per_b,
                          pl.ds(pl.multiple_of((i % chunks_per_b) * _CR, _CR),
                                _CR)]

    def sig_out_chunk(i):
        return sig_out_hbm.at[i // chunks_per_b,
                              pl.ds(pl.multiple_of((i % chunks_per_b) * _CR,
                                                   _CR), _CR)]

    pltpu.make_async_copy(mur_hbm, mur_v, small_sems.at[0]).start()
    pltpu.make_async_copy(keepr_hbm, keepr_v, small_sems.at[1]).start()

    for d in range(_DEPTH):
        pltpu.make_async_copy(sig_in_chunk(d), in_bufs.at[d],
                              in_sems.at[d]).start()

    pltpu.make_async_copy(mur_hbm, mur_v, small_sems.at[0]).wait()
    pltpu.make_async_copy(keepr_hbm, keepr_v, small_sems.at[1]).wait()
    mur_v[...] = mur_v[...] * keepr_v[...]
    pltpu.make_async_copy(mur_v, mu_out_hbm, small_sems.at[0]).start()

    for b in range(n_b):
        nzb = jnp.where(mur_v[b].reshape(chunks_per_b * _CR, -1) != 0.0,
                        1.0, 0.0)
        colf_v[b] = nzb.swapaxes(0, 1)

    def loop(i, carry):
        slot = lax.rem(i, _DEPTH)
        b = i // chunks_per_b
        ic = lax.rem(i, chunks_per_b)
        pltpu.make_async_copy(sig_in_chunk(i), in_bufs.at[slot],
                              in_sems.at[slot]).wait()

        @pl.when(i >= _DEPTH)
        def _():
            pltpu.make_async_copy(out_bufs.at[slot], sig_out_chunk(i),
                                  out_sems.at[slot]).wait()

        sig = in_bufs[slot]
        rowf = jnp.where(mur_v[b, ic] != 0.0, _S2, 0.0)
        colf = colf_v[b]
        out_bufs[slot] = sig * rowf[:, :, None] * colf[None, :, :]
        pltpu.make_async_copy(out_bufs.at[slot], sig_out_chunk(i),
                              out_sems.at[slot]).start()

        @pl.when(i + _DEPTH < n_chunks)
        def _():
            pltpu.make_async_copy(sig_in_chunk(i + _DEPTH), in_bufs.at[slot],
                                  in_sems.at[slot]).start()
        return carry

    lax.fori_loop(0, n_chunks, loop, 0)

    for d in range(_DEPTH):
        i = n_chunks - _DEPTH + d
        slot = i % _DEPTH
        pltpu.make_async_copy(out_bufs.at[slot], sig_out_chunk(i),
                              out_sems.at[slot]).wait()
    pltpu.make_async_copy(mur_v, mu_out_hbm, small_sems.at[0]).wait()


def kernel(mu_in, Sigma_in):
    B, H, W, C = mu_in.shape
    HW = H * W
    chunks_per_b = HW // _CR
    (keep_r_np,) = _keep_masks((B, H, W, C))
    keep_r = jnp.asarray(keep_r_np)
    mu_r = mu_in.reshape(B, chunks_per_b, _CR, C)
    sig_t = jnp.transpose(Sigma_in, (0, 1, 3, 2))

    hbm = pl.BlockSpec(memory_space=pltpu.MemorySpace.HBM)
    mu_out_r, sig_out_t = pl.pallas_call(
        _body,
        in_specs=[hbm] * 3,
        out_specs=[hbm, hbm],
        out_shape=[
            jax.ShapeDtypeStruct((B, chunks_per_b, _CR, C), jnp.float32),
            jax.ShapeDtypeStruct((B, HW, C, HW), jnp.float32),
        ],
        scratch_shapes=[
            pltpu.VMEM((B, chunks_per_b, _CR, C), jnp.float32),
            pltpu.VMEM((B, chunks_per_b, _CR, C), jnp.float32),
            pltpu.VMEM((B, C, HW), jnp.float32),
            pltpu.VMEM((_DEPTH, _CR, C, HW), jnp.float32),
            pltpu.VMEM((_DEPTH, _CR, C, HW), jnp.float32),
            pltpu.SemaphoreType.DMA((2,)),
            pltpu.SemaphoreType.DMA((_DEPTH,)),
            pltpu.SemaphoreType.DMA((_DEPTH,)),
        ],
    )(mu_r, keep_r, sig_t)

    mu_out = mu_out_r.reshape(B, H, W, C)
    sig_out = jnp.transpose(sig_out_t, (0, 1, 3, 2))
    return mu_out, sig_out
